# baseline (speedup 1.0000x reference)
import jax
import jax.numpy as jnp
from jax import lax
from jax.experimental import pallas as pl

N = 10000
E = 320000
D = 128
K = 512
B = 64
ALPHA = 0.5
GAMMA = 0.5
CW = 0.25
NL = 3

RBLK = 2000
NBLKS = N // RBLK


def _layer_body(hs, he, ags, age, sW, sb, eW, eb, os_, oe_):
    os_[...] = jnp.maximum(
        jnp.dot(hs[...] + ags[...], sW[...], preferred_element_type=jnp.float32)
        + sb[...], 0.0)
    oe_[...] = jnp.maximum(
        jnp.dot(he[...] + age[...], eW[...], preferred_element_type=jnp.float32)
        + eb[...], 0.0)


def _layer_update(hs, he, ags, age, sW, sb, eW, eb):
    row = pl.BlockSpec((RBLK, D), lambda i: (i, 0))
    wspec = pl.BlockSpec((D, D), lambda i: (0, 0))
    bspec = pl.BlockSpec((1, D), lambda i: (0, 0))
    return pl.pallas_call(
        _layer_body,
        grid=(NBLKS,),
        in_specs=[row, row, row, row, wspec, bspec, wspec, bspec],
        out_specs=[row, row],
        out_shape=[jax.ShapeDtypeStruct((N, D), jnp.float32)] * 2,
    )(hs, he, ags, age, sW, sb, eW, eb)


def _h1_body(sfeat, W1, b1, o):
    o[...] = jnp.dot(sfeat[...], W1[...],
                     preferred_element_type=jnp.float32) + b1[...]


def _h1_pre(sfeat, W1, b1):
    return pl.pallas_call(
        _h1_body,
        grid=(NBLKS,),
        in_specs=[
            pl.BlockSpec((RBLK, D), lambda i: (i, 0)),
            pl.BlockSpec((D, 2 * D), lambda i: (0, 0)),
            pl.BlockSpec((1, 2 * D), lambda i: (0, 0)),
        ],
        out_specs=pl.BlockSpec((RBLK, 2 * D), lambda i: (i, 0)),
        out_shape=jax.ShapeDtypeStruct((N, 2 * D), jnp.float32),
    )(sfeat, W1, b1)


def _score_body(h1pre, nf, mu, var, g, be, W2, b2, score_o, hc_o, hs_o):
    h1 = g[...] * (h1pre[...] - mu[...]) / jnp.sqrt(var[...] + 1e-5) + be[...]
    h1 = jnp.maximum(h1, 0.0)
    score = jax.nn.sigmoid(
        jnp.dot(h1, W2[...], preferred_element_type=jnp.float32) + b2[...])
    nfv = nf[...]
    score_o[...] = score
    hc_o[...] = nfv * score + (ALPHA * nfv) * (1.0 - score)
    hs_o[...] = nfv * (1.0 - score) + (ALPHA * nfv) * score


def _score_pass(h1pre, nf, mu, var, p):
    row = pl.BlockSpec((RBLK, D), lambda i: (i, 0))
    row2 = pl.BlockSpec((RBLK, 2 * D), lambda i: (i, 0))
    full = lambda r, c: pl.BlockSpec((r, c), lambda i: (0, 0))
    return pl.pallas_call(
        _score_body,
        grid=(NBLKS,),
        in_specs=[row2, row, full(1, 2 * D), full(1, 2 * D), full(1, 2 * D),
                  full(1, 2 * D), full(2 * D, D), full(1, D)],
        out_specs=[row, row, row],
        out_shape=[jax.ShapeDtypeStruct((N, D), jnp.float32)] * 3,
    )(h1pre, nf, mu.reshape(1, 2 * D), var.reshape(1, 2 * D),
      p["sepg"].reshape(1, -1), p["sepbe"].reshape(1, -1),
      p["sepW2"], p["sepb2"].reshape(1, -1))


def _vq_q(h, hh, cbT, cb, cb2):
    d2 = hh - 2.0 * jnp.dot(h, cbT, preferred_element_type=jnp.float32) + cb2
    m = jnp.min(d2, axis=1, keepdims=True)
    iota = lax.broadcasted_iota(jnp.int32, (h.shape[0], K), 1)
    idx = jnp.min(jnp.where(d2 == m, iota, K), axis=1)
    ohk = (iota == idx[:, None]).astype(jnp.float32)
    q = jnp.dot(ohk, cb, preferred_element_type=jnp.float32,
                precision=lax.Precision.HIGHEST)
    return q


def _main_body(nf, score, Hc, Hs, hhc, hhs, batch,
               cbcT, cbc, cbc2, cbsT, cbs, cbs2,
               c_node, s_node, pool_c, pool_s, pos_o, neg_o, cnt_o,
               csq_o, ssq_o):
    nfv = nf[...]
    sc = score[...]
    hc = Hc[...]
    hs = Hs[...]
    qc = _vq_q(hc, hhc[...], cbcT[...], cbc[...], cbc2[...])
    qs = _vq_q(hs, hhs[...], cbsT[...], cbs[...], cbs2[...])
    qc = hc + (qc - hc)
    qs = hs + (qs - hs)
    cn = nfv * sc + qc
    sn = nfv * (1.0 - sc) + qs
    c_node[...] = cn
    s_node[...] = sn

    pos_node = jnp.mean(sc, axis=1, keepdims=True)
    oh = (batch[...] == lax.broadcasted_iota(jnp.int32, (RBLK, B), 1)
          ).astype(jnp.float32)
    dn = (((0,), (0,)), ((), ()))
    p_c = lax.dot_general(oh, cn, dn, preferred_element_type=jnp.float32,
                          precision=lax.Precision.HIGHEST)
    p_s = lax.dot_general(oh, sn, dn, preferred_element_type=jnp.float32,
                          precision=lax.Precision.HIGHEST)
    posv = jnp.sum(oh * pos_node, axis=0, keepdims=True)
    negv = jnp.sum(oh * (1.0 - pos_node), axis=0, keepdims=True)
    cntv = jnp.sum(oh, axis=0, keepdims=True)
    csq = jnp.sum((qc - hc) ** 2).reshape(1, 1)
    ssq = jnp.sum((qs - hs) ** 2).reshape(1, 1)

    @pl.when(pl.program_id(0) == 0)
    def _():
        pool_c[...] = jnp.zeros_like(pool_c)
        pool_s[...] = jnp.zeros_like(pool_s)
        pos_o[...] = jnp.zeros_like(pos_o)
        neg_o[...] = jnp.zeros_like(neg_o)
        cnt_o[...] = jnp.zeros_like(cnt_o)
        csq_o[...] = jnp.zeros_like(csq_o)
        ssq_o[...] = jnp.zeros_like(ssq_o)

    pool_c[...] += p_c
    pool_s[...] += p_s
    pos_o[...] += posv
    neg_o[...] += negv
    cnt_o[...] += cntv
    csq_o[...] += csq
    ssq_o[...] += ssq


def _main_pass(nf, score, Hc, Hs, hhc, hhs, batch2d, p, cbc2, cbs2):
    row = pl.BlockSpec((RBLK, D), lambda i: (i, 0))
    col = pl.BlockSpec((RBLK, 1), lambda i: (i, 0))
    full = lambda r, c: pl.BlockSpec((r, c), lambda i: (0, 0))
    cbc = p["cbc"]
    cbs = p["cbs"]
    return pl.pallas_call(
        _main_body,
        grid=(NBLKS,),
        in_specs=[
            row, row, row, row, col, col, col,
            full(D, K), full(K, D), full(1, K),
            full(D, K), full(K, D), full(1, K),
        ],
        out_specs=[
            row, row,
            full(B, D), full(B, D),
            full(1, B), full(1, B), full(1, B),
            full(1, 1), full(1, 1),
        ],
        out_shape=[
            jax.ShapeDtypeStruct((N, D), jnp.float32),
            jax.ShapeDtypeStruct((N, D), jnp.float32),
            jax.ShapeDtypeStruct((B, D), jnp.float32),
            jax.ShapeDtypeStruct((B, D), jnp.float32),
            jax.ShapeDtypeStruct((1, B), jnp.float32),
            jax.ShapeDtypeStruct((1, B), jnp.float32),
            jax.ShapeDtypeStruct((1, B), jnp.float32),
            jax.ShapeDtypeStruct((1, 1), jnp.float32),
            jax.ShapeDtypeStruct((1, 1), jnp.float32),
        ],
    )(nf, score, Hc, Hs, hhc, hhs, batch2d,
      cbc.T, cbc, cbc2, cbs.T, cbs, cbs2)


def _final_body(pool_c, pool_s, pos, neg, cnt, csq, ssq,
                W1, b1, g, be, W2, b2,
                c_logit, c_graph, s_graph, cl_c, cl_s, loss_reg):
    cntc = jnp.maximum(cnt[...], 1.0)
    cg = pool_c[...] / cntc.reshape(B, 1)
    sg = pool_s[...] / cntc.reshape(B, 1)
    c_graph[...] = cg
    s_graph[...] = sg
    h2 = jnp.dot(cg, W1[...], preferred_element_type=jnp.float32) + b1[...]
    mu = jnp.mean(h2, axis=0, keepdims=True)
    var = jnp.mean((h2 - mu) ** 2, axis=0, keepdims=True)
    h2 = g[...] * (h2 - mu) / jnp.sqrt(var + 1e-5) + be[...]
    h2 = jnp.maximum(h2, 0.0)
    c_logit[...] = jnp.dot(h2, W2[...], preferred_element_type=jnp.float32) + b2[...]
    posv = pos[...] + 1e-8
    negv = neg[...] + 1e-8
    loss_reg[...] = jnp.mean(jnp.abs(posv / (posv + negv) - GAMMA)).reshape(1, 1)
    cl_c[...] = CW * (csq[...] / (N * D))
    cl_s[...] = CW * (ssq[...] / (N * D))


def _finalize(pool_c, pool_s, pos, neg, cnt, csq, ssq, p):
    full = lambda r, c: pl.BlockSpec((r, c), lambda: (0, 0))
    return pl.pallas_call(
        _final_body,
        in_specs=[
            full(B, D), full(B, D), full(1, B), full(1, B), full(1, B),
            full(1, 1), full(1, 1),
            full(D, 2 * D), full(1, 2 * D), full(1, 2 * D), full(1, 2 * D),
            full(2 * D, 1), full(1, 1),
        ],
        out_specs=[
            full(B, 1), full(B, D), full(B, D),
            full(1, 1), full(1, 1), full(1, 1),
        ],
        out_shape=[
            jax.ShapeDtypeStruct((B, 1), jnp.float32),
            jax.ShapeDtypeStruct((B, D), jnp.float32),
            jax.ShapeDtypeStruct((B, D), jnp.float32),
            jax.ShapeDtypeStruct((1, 1), jnp.float32),
            jax.ShapeDtypeStruct((1, 1), jnp.float32),
            jax.ShapeDtypeStruct((1, 1), jnp.float32),
        ],
    )(pool_c, pool_s, pos, neg, cnt, csq, ssq,
      p["clsW1"], p["clsb1"].reshape(1, -1), p["clsg"].reshape(1, -1),
      p["clsbe"].reshape(1, -1), p["clsW2"], p["clsb2"].reshape(1, -1))


def _edge_agg(h, src, dst):
    return jax.ops.segment_sum(h[src], dst, num_segments=N)


def kernel(x, edge_index, batch, params):
    p = params
    src = edge_index[0]
    dst = edge_index[1]

    a0 = _edge_agg(x, src, dst)
    hs, he = _layer_update(x, x, a0, a0,
                           p["sW0"], p["sb0"].reshape(1, -1),
                           p["eW0"], p["eb0"].reshape(1, -1))
    for l in (1, 2):
        ags = _edge_agg(hs, src, dst)
        age = _edge_agg(he, src, dst)
        hs, he = _layer_update(hs, he, ags, age,
                               p["sW" + str(l)], p["sb" + str(l)].reshape(1, -1),
                               p["eW" + str(l)], p["eb" + str(l)].reshape(1, -1))

    h1pre = _h1_pre(hs, p["sepW1"], p["sepb1"].reshape(1, -1))
    h1stat = hs @ p["sepW1"] + p["sepb1"]
    mu = jnp.mean(h1stat, axis=0)
    var = jnp.var(h1stat, axis=0)
    score, Hc, Hs = _score_pass(h1pre, he, mu, var, p)
    hhc = jnp.sum(Hc * Hc, axis=1, keepdims=True)
    hhs = jnp.sum(Hs * Hs, axis=1, keepdims=True)
    cbc2 = jnp.sum(p["cbc"] * p["cbc"], axis=1).reshape(1, K)
    cbs2 = jnp.sum(p["cbs"] * p["cbs"], axis=1).reshape(1, K)

    batch2d = batch.reshape(N, 1)
    (c_node, s_node, pool_c, pool_s, pos, neg, cnt,
     csq, ssq) = _main_pass(he, score, Hc, Hs, hhc, hhs, batch2d, p, cbc2, cbs2)
    c_logit, c_graph, s_graph, cl_c, cl_s, loss_reg = _finalize(
        pool_c, pool_s, pos, neg, cnt, csq, ssq, p)
    return (c_logit, c_graph, s_graph,
            cl_c.reshape(()), cl_s.reshape(()), loss_reg.reshape(()),
            c_node, s_node)


# speedup vs baseline: 1.0021x; 1.0021x over previous
import jax
import jax.numpy as jnp
from jax import lax
from jax.experimental import pallas as pl

N = 10000
E = 320000
D = 128
K = 512
B = 64
ALPHA = 0.5
GAMMA = 0.5
CW = 0.25
NL = 3

RBLK = 2000
NBLKS = N // RBLK


def _layer_body(hs, he, ags, age, sW, sb, eW, eb, os_, oe_):
    os_[...] = jnp.maximum(
        jnp.dot(hs[...] + ags[...], sW[...], preferred_element_type=jnp.float32)
        + sb[...], 0.0)
    oe_[...] = jnp.maximum(
        jnp.dot(he[...] + age[...], eW[...], preferred_element_type=jnp.float32)
        + eb[...], 0.0)


def _layer_update(hs, he, ags, age, sW, sb, eW, eb):
    row = pl.BlockSpec((RBLK, D), lambda i: (i, 0))
    wspec = pl.BlockSpec((D, D), lambda i: (0, 0))
    bspec = pl.BlockSpec((1, D), lambda i: (0, 0))
    return pl.pallas_call(
        _layer_body,
        grid=(NBLKS,),
        in_specs=[row, row, row, row, wspec, bspec, wspec, bspec],
        out_specs=[row, row],
        out_shape=[jax.ShapeDtypeStruct((N, D), jnp.float32)] * 2,
    )(hs, he, ags, age, sW, sb, eW, eb)


def _score_body(h1pre, mu, var, g, be, W2, b2, score_o):
    h1 = g[...] * (h1pre[...] - mu[...]) / jnp.sqrt(var[...] + 1e-5) + be[...]
    h1 = jnp.maximum(h1, 0.0)
    score_o[...] = jax.nn.sigmoid(
        jnp.dot(h1, W2[...], preferred_element_type=jnp.float32) + b2[...])


def _score_pass(h1pre, mu, var, p):
    row = pl.BlockSpec((RBLK, D), lambda i: (i, 0))
    row2 = pl.BlockSpec((RBLK, 2 * D), lambda i: (i, 0))
    full = lambda r, c: pl.BlockSpec((r, c), lambda i: (0, 0))
    return pl.pallas_call(
        _score_body,
        grid=(NBLKS,),
        in_specs=[row2, full(1, 2 * D), full(1, 2 * D), full(1, 2 * D),
                  full(1, 2 * D), full(2 * D, D), full(1, D)],
        out_specs=row,
        out_shape=jax.ShapeDtypeStruct((N, D), jnp.float32),
    )(h1pre, mu.reshape(1, 2 * D), var.reshape(1, 2 * D),
      p["sepg"].reshape(1, -1), p["sepbe"].reshape(1, -1),
      p["sepW2"], p["sepb2"].reshape(1, -1))


def _vq_q(h, hh, cbT, cb, cb2):
    d2 = hh - 2.0 * jnp.dot(h, cbT, preferred_element_type=jnp.float32) + cb2
    m = jnp.min(d2, axis=1, keepdims=True)
    iota = lax.broadcasted_iota(jnp.int32, (h.shape[0], K), 1)
    idx = jnp.min(jnp.where(d2 == m, iota, K), axis=1)
    ohk = (iota == idx[:, None]).astype(jnp.float32)
    q = jnp.dot(ohk, cb, preferred_element_type=jnp.float32,
                precision=lax.Precision.HIGHEST)
    return q


def _main_body(nf, score, hhc, hhs, batch,
               cbcT, cbc, cbc2, cbsT, cbs, cbs2,
               c_node, s_node, pool_c, pool_s, pos_o, neg_o, cnt_o,
               csq_o, ssq_o):
    nfv = nf[...]
    sc = score[...]
    hc = nfv * sc + (ALPHA * nfv) * (1.0 - sc)
    hs = nfv * (1.0 - sc) + (ALPHA * nfv) * sc
    qc = _vq_q(hc, hhc[...], cbcT[...], cbc[...], cbc2[...])
    qs = _vq_q(hs, hhs[...], cbsT[...], cbs[...], cbs2[...])
    qc = hc + (qc - hc)
    qs = hs + (qs - hs)
    cn = nfv * sc + qc
    sn = nfv * (1.0 - sc) + qs
    c_node[...] = cn
    s_node[...] = sn

    pos_node = jnp.mean(sc, axis=1, keepdims=True)
    oh = (batch[...] == lax.broadcasted_iota(jnp.int32, (RBLK, B), 1)
          ).astype(jnp.float32)
    dn = (((0,), (0,)), ((), ()))
    p_c = lax.dot_general(oh, cn, dn, preferred_element_type=jnp.float32,
                          precision=lax.Precision.HIGHEST)
    p_s = lax.dot_general(oh, sn, dn, preferred_element_type=jnp.float32,
                          precision=lax.Precision.HIGHEST)
    posv = jnp.sum(oh * pos_node, axis=0, keepdims=True)
    negv = jnp.sum(oh * (1.0 - pos_node), axis=0, keepdims=True)
    cntv = jnp.sum(oh, axis=0, keepdims=True)
    csq = jnp.sum((qc - hc) ** 2).reshape(1, 1)
    ssq = jnp.sum((qs - hs) ** 2).reshape(1, 1)

    @pl.when(pl.program_id(0) == 0)
    def _():
        pool_c[...] = jnp.zeros_like(pool_c)
        pool_s[...] = jnp.zeros_like(pool_s)
        pos_o[...] = jnp.zeros_like(pos_o)
        neg_o[...] = jnp.zeros_like(neg_o)
        cnt_o[...] = jnp.zeros_like(cnt_o)
        csq_o[...] = jnp.zeros_like(csq_o)
        ssq_o[...] = jnp.zeros_like(ssq_o)

    pool_c[...] += p_c
    pool_s[...] += p_s
    pos_o[...] += posv
    neg_o[...] += negv
    cnt_o[...] += cntv
    csq_o[...] += csq
    ssq_o[...] += ssq


def _main_pass(nf, score, hhc, hhs, batch2d, p, cbc2, cbs2):
    row = pl.BlockSpec((RBLK, D), lambda i: (i, 0))
    col = pl.BlockSpec((RBLK, 1), lambda i: (i, 0))
    full = lambda r, c: pl.BlockSpec((r, c), lambda i: (0, 0))
    cbc = p["cbc"]
    cbs = p["cbs"]
    return pl.pallas_call(
        _main_body,
        grid=(NBLKS,),
        in_specs=[
            row, row, col, col, col,
            full(D, K), full(K, D), full(1, K),
            full(D, K), full(K, D), full(1, K),
        ],
        out_specs=[
            row, row,
            full(B, D), full(B, D),
            full(1, B), full(1, B), full(1, B),
            full(1, 1), full(1, 1),
        ],
        out_shape=[
            jax.ShapeDtypeStruct((N, D), jnp.float32),
            jax.ShapeDtypeStruct((N, D), jnp.float32),
            jax.ShapeDtypeStruct((B, D), jnp.float32),
            jax.ShapeDtypeStruct((B, D), jnp.float32),
            jax.ShapeDtypeStruct((1, B), jnp.float32),
            jax.ShapeDtypeStruct((1, B), jnp.float32),
            jax.ShapeDtypeStruct((1, B), jnp.float32),
            jax.ShapeDtypeStruct((1, 1), jnp.float32),
            jax.ShapeDtypeStruct((1, 1), jnp.float32),
        ],
    )(nf, score, hhc, hhs, batch2d,
      cbc.T, cbc, cbc2, cbs.T, cbs, cbs2)


def _final_body(pool_c, pool_s, pos, neg, cnt, csq, ssq,
                W1, b1, g, be, W2, b2,
                c_logit, c_graph, s_graph, cl_c, cl_s, loss_reg):
    cntc = jnp.maximum(cnt[...], 1.0)
    cg = pool_c[...] / cntc.reshape(B, 1)
    sg = pool_s[...] / cntc.reshape(B, 1)
    c_graph[...] = cg
    s_graph[...] = sg
    h2 = jnp.dot(cg, W1[...], preferred_element_type=jnp.float32) + b1[...]
    mu = jnp.mean(h2, axis=0, keepdims=True)
    var = jnp.mean((h2 - mu) ** 2, axis=0, keepdims=True)
    h2 = g[...] * (h2 - mu) / jnp.sqrt(var + 1e-5) + be[...]
    h2 = jnp.maximum(h2, 0.0)
    c_logit[...] = jnp.dot(h2, W2[...], preferred_element_type=jnp.float32) + b2[...]
    posv = pos[...] + 1e-8
    negv = neg[...] + 1e-8
    loss_reg[...] = jnp.mean(jnp.abs(posv / (posv + negv) - GAMMA)).reshape(1, 1)
    cl_c[...] = CW * (csq[...] / (N * D))
    cl_s[...] = CW * (ssq[...] / (N * D))


def _finalize(pool_c, pool_s, pos, neg, cnt, csq, ssq, p):
    full = lambda r, c: pl.BlockSpec((r, c), lambda: (0, 0))
    return pl.pallas_call(
        _final_body,
        in_specs=[
            full(B, D), full(B, D), full(1, B), full(1, B), full(1, B),
            full(1, 1), full(1, 1),
            full(D, 2 * D), full(1, 2 * D), full(1, 2 * D), full(1, 2 * D),
            full(2 * D, 1), full(1, 1),
        ],
        out_specs=[
            full(B, 1), full(B, D), full(B, D),
            full(1, 1), full(1, 1), full(1, 1),
        ],
        out_shape=[
            jax.ShapeDtypeStruct((B, 1), jnp.float32),
            jax.ShapeDtypeStruct((B, D), jnp.float32),
            jax.ShapeDtypeStruct((B, D), jnp.float32),
            jax.ShapeDtypeStruct((1, 1), jnp.float32),
            jax.ShapeDtypeStruct((1, 1), jnp.float32),
            jax.ShapeDtypeStruct((1, 1), jnp.float32),
        ],
    )(pool_c, pool_s, pos, neg, cnt, csq, ssq,
      p["clsW1"], p["clsb1"].reshape(1, -1), p["clsg"].reshape(1, -1),
      p["clsbe"].reshape(1, -1), p["clsW2"], p["clsb2"].reshape(1, -1))


def _edge_agg(h, src, dst):
    return jax.ops.segment_sum(h[src], dst, num_segments=N)


def kernel(x, edge_index, batch, params):
    p = params
    src = edge_index[0]
    dst = edge_index[1]

    a0 = _edge_agg(x, src, dst)
    hs, he = _layer_update(x, x, a0, a0,
                           p["sW0"], p["sb0"].reshape(1, -1),
                           p["eW0"], p["eb0"].reshape(1, -1))
    for l in (1, 2):
        ags = _edge_agg(hs, src, dst)
        age = _edge_agg(he, src, dst)
        hs, he = _layer_update(hs, he, ags, age,
                               p["sW" + str(l)], p["sb" + str(l)].reshape(1, -1),
                               p["eW" + str(l)], p["eb" + str(l)].reshape(1, -1))

    h1pre = hs @ p["sepW1"] + p["sepb1"]
    mu = jnp.mean(h1pre, axis=0)
    var = jnp.var(h1pre, axis=0)
    score = _score_pass(h1pre, mu, var, p)
    Hcx = he * score + ALPHA * he * (1.0 - score)
    Hsx = he * (1.0 - score) + ALPHA * he * score
    hhc = jnp.sum(Hcx * Hcx, axis=1, keepdims=True)
    hhs = jnp.sum(Hsx * Hsx, axis=1, keepdims=True)
    cbc2 = jnp.sum(p["cbc"] * p["cbc"], axis=1).reshape(1, K)
    cbs2 = jnp.sum(p["cbs"] * p["cbs"], axis=1).reshape(1, K)

    batch2d = batch.reshape(N, 1)
    (c_node, s_node, pool_c, pool_s, pos, neg, cnt,
     csq, ssq) = _main_pass(he, score, hhc, hhs, batch2d, p, cbc2, cbs2)
    c_logit, c_graph, s_graph, cl_c, cl_s, loss_reg = _finalize(
        pool_c, pool_s, pos, neg, cnt, csq, ssq, p)
    return (c_logit, c_graph, s_graph,
            cl_c.reshape(()), cl_s.reshape(()), loss_reg.reshape(()),
            c_node, s_node)
